# baseline (speedup 1.0000x reference)
import functools

import jax
import jax.numpy as jnp
from jax import lax
from jax.experimental import pallas as pl
from jax.experimental.pallas import tpu as pltpu
from jax.experimental.pallas import tpu_sc as plsc

N = 10000
NP = 10240
E = 160000
CHUNK = 128
TILES = 16
AGG_CH = 80
DEG_CH = 40
ROWS = TILES * AGG_CH
ECAP = ROWS * CHUNK
NB = 2

_f32 = jnp.float32
_SDS = jax.ShapeDtypeStruct


def _mesh():
    return plsc.VectorSubcoreMesh(core_axis_name="c", subcore_axis_name="s")


def _deg_call(dst_rows, ones_sm, zeros_sm):
    @functools.partial(
        pl.kernel,
        out_type=(_SDS((NP, 128), _f32), _SDS((NP, 128), _f32)),
        scratch_types=[
            pltpu.VMEM((DEG_CH, CHUNK), jnp.int32),
            pltpu.VMEM((CHUNK, 128), _f32),
            pltpu.VMEM_SHARED((NP, 128), _f32),
        ],
        mesh=_mesh(),
    )
    def deg_k(dst_hbm, ones_hbm, z_hbm, cnt0, cnt1, idx_v, ones_v, acc):
        cid = lax.axis_index("c")
        sid = lax.axis_index("s")
        rows_per_tile = NP // TILES
        rbase = sid * rows_per_tile
        pltpu.sync_copy(z_hbm.at[pl.ds(rbase, rows_per_tile)],
                        acc.at[pl.ds(rbase, rows_per_tile)])
        pltpu.sync_copy(ones_hbm, ones_v)
        ebase = cid * (TILES * DEG_CH) + sid * DEG_CH
        pltpu.sync_copy(dst_hbm.at[pl.ds(ebase, DEG_CH)], idx_v)
        plsc.subcore_barrier()

        @pl.loop(0, DEG_CH)
        def _(j):
            pltpu.sync_copy(ones_v, acc.at[idx_v.at[j]], add=True)

        plsc.subcore_barrier()

        @pl.when(cid == 0)
        def _():
            pltpu.sync_copy(acc.at[pl.ds(rbase, rows_per_tile)],
                            cnt0.at[pl.ds(rbase, rows_per_tile)])

        @pl.when(cid == 1)
        def _():
            pltpu.sync_copy(acc.at[pl.ds(rbase, rows_per_tile)],
                            cnt1.at[pl.ds(rbase, rows_per_tile)])

    return deg_k(dst_rows, ones_sm, zeros_sm)


def _agg_call(s0, s1, src_rows, dst_rows, zeros_big):
    @functools.partial(
        pl.kernel,
        out_type=(_SDS((NP, 128), _f32), _SDS((NP, 128), _f32)),
        scratch_types=[
            pltpu.VMEM((AGG_CH, CHUNK // 2), jnp.int32),
            pltpu.VMEM((AGG_CH // 2, CHUNK), jnp.int32),
            pltpu.VMEM((NB, CHUNK, 128), _f32),
            pltpu.SemaphoreType.DMA((NB, 2)),
            pltpu.SemaphoreType.DMA((NB,)),
            pltpu.VMEM_SHARED((NP, 128), _f32),
        ],
        mesh=_mesh(),
    )
    def agg_k(s0_hbm, s1_hbm, sp_hbm, dp_hbm, z_hbm,
              out0, out1, sidx, didx, bufs, sem_g, sem_s, acc):
        cid = lax.axis_index("c")
        sid = lax.axis_index("s")
        rows_per_tile = NP // TILES
        rbase = sid * rows_per_tile
        ebase = sid * AGG_CH
        gbase = sid * (AGG_CH * 2)

        def pipeline(s_hbm, out_hbm):
            HC = AGG_CH // 2
            GH = AGG_CH

            def start_gathers(j, b):
                for q in range(2):
                    pltpu.async_copy(
                        s_hbm.at[sidx.at[2 * j + q]],
                        bufs.at[b, pl.ds(q * (CHUNK // 2), CHUNK // 2)],
                        sem_g.at[b, q])

            def wait_gathers(b):
                for q in range(2):
                    pltpu.make_async_copy(
                        s_hbm.at[sidx.at[q]],
                        bufs.at[b, pl.ds(q * (CHUNK // 2), CHUNK // 2)],
                        sem_g.at[b, q]).wait()

            pltpu.sync_copy(sp_hbm.at[pl.ds(gbase, GH)], sidx)
            pltpu.sync_copy(dp_hbm.at[pl.ds(ebase, HC)], didx)
            for h in range(2):
                if h:
                    for b in range(NB):
                        pltpu.make_async_copy(
                            bufs.at[b], acc.at[didx.at[b]],
                            sem_s.at[b]).wait()
                    pltpu.sync_copy(sp_hbm.at[pl.ds(gbase + GH, GH)], sidx)
                    pltpu.sync_copy(dp_hbm.at[pl.ds(ebase + HC, HC)], didx)
                for b in range(NB):
                    start_gathers(b, b)
                if h == 0:
                    plsc.subcore_barrier()

                @pl.loop(0, HC // NB)
                def _(g):
                    for b in range(NB):
                        j = g * NB + b
                        wait_gathers(b)
                        pltpu.async_copy(bufs.at[b], acc.at[didx.at[j]],
                                        sem_s.at[b], add=True)
                        nxt = j + NB

                        @pl.when(nxt < HC)
                        def _():
                            pltpu.make_async_copy(
                                bufs.at[b], acc.at[didx.at[j]],
                                sem_s.at[b]).wait()
                            start_gathers(nxt, b)

            for b in range(NB):
                pltpu.make_async_copy(
                    bufs.at[b], acc.at[didx.at[b]], sem_s.at[b]).wait()
            plsc.subcore_barrier()
            pltpu.sync_copy(acc.at[pl.ds(rbase, rows_per_tile)],
                            out_hbm.at[pl.ds(rbase, rows_per_tile)])

        pltpu.sync_copy(z_hbm.at[pl.ds(rbase, rows_per_tile)],
                        acc.at[pl.ds(rbase, rows_per_tile)])

        @pl.when(cid == 0)
        def _():
            pipeline(s0_hbm, out0)

        @pl.when(cid == 1)
        def _():
            pipeline(s1_hbm, out1)

    return agg_k(s0, s1, src_rows, dst_rows, zeros_big)


def _k1_body(x_ref, c0_ref, c1_ref, s0_ref, s1_ref, dinv_ref):
    cnt = c0_ref[...][:, :16] + c1_ref[...][:, :16]
    dinv16 = lax.rsqrt(cnt + 1.0)
    dinv = dinv16[:, :1]
    s = x_ref[...] * dinv
    s0_ref[...] = s[:, :128]
    s1_ref[...] = s[:, 128:]
    dinv_ref[...] = dinv16


def _k1(x_pad, cnt0, cnt1):
    R = 256
    return pl.pallas_call(
        _k1_body,
        grid=(NP // R,),
        in_specs=[pl.BlockSpec((R, 256), lambda i: (i, 0)),
                  pl.BlockSpec((R, 128), lambda i: (i, 0)),
                  pl.BlockSpec((R, 128), lambda i: (i, 0))],
        out_specs=[pl.BlockSpec((R, 128), lambda i: (i, 0)),
                   pl.BlockSpec((R, 128), lambda i: (i, 0)),
                   pl.BlockSpec((R, 16), lambda i: (i, 0))],
        out_shape=[_SDS((NP, 128), _f32), _SDS((NP, 128), _f32),
                   _SDS((NP, 16), _f32)],
    )(x_pad, cnt0, cnt1)


def _k2_body(a0_ref, a1_ref, x_ref, dinv_ref, w1_ref, b1_ref, w2_ref,
             g_ref, t0_ref, t1_ref):
    dinv = dinv_ref[...][:, :1]
    a = jnp.concatenate([a0_ref[...], a1_ref[...]], axis=1)
    t = dinv * a + (dinv * dinv) * x_ref[...]
    h1 = jnp.maximum(
        jnp.dot(t, w1_ref[...], preferred_element_type=_f32) + b1_ref[...],
        0.0)
    g = jnp.dot(h1, w2_ref[...], preferred_element_type=_f32)
    g_ref[...] = g
    s2 = g * dinv
    t0_ref[...] = s2[:, :128]
    t1_ref[...] = s2[:, 128:]


def _k2(a0, a1, x_pad, dinv16, W1, b1r, W2):
    R = 256
    return pl.pallas_call(
        _k2_body,
        grid=(NP // R,),
        in_specs=[pl.BlockSpec((R, 128), lambda i: (i, 0)),
                  pl.BlockSpec((R, 128), lambda i: (i, 0)),
                  pl.BlockSpec((R, 256), lambda i: (i, 0)),
                  pl.BlockSpec((R, 16), lambda i: (i, 0)),
                  pl.BlockSpec((256, 512), lambda i: (0, 0)),
                  pl.BlockSpec((1, 512), lambda i: (0, 0)),
                  pl.BlockSpec((512, 256), lambda i: (0, 0))],
        out_specs=[pl.BlockSpec((R, 256), lambda i: (i, 0)),
                   pl.BlockSpec((R, 128), lambda i: (i, 0)),
                   pl.BlockSpec((R, 128), lambda i: (i, 0))],
        out_shape=[_SDS((NP, 256), _f32), _SDS((NP, 128), _f32),
                   _SDS((NP, 128), _f32)],
    )(a0, a1, x_pad, dinv16, W1, b1r, W2)


def _k3_body(a0_ref, a1_ref, g_ref, dinv_ref, b2_ref, out_ref):
    dinv = dinv_ref[...][:, :1]
    a = jnp.concatenate([a0_ref[...], a1_ref[...]], axis=1)
    out_ref[...] = dinv * a + (dinv * dinv) * g_ref[...] + b2_ref[...]


def _k3(a0, a1, g, dinv16, b2r):
    R = 400
    return pl.pallas_call(
        _k3_body,
        grid=(N // R,),
        in_specs=[pl.BlockSpec((R, 128), lambda i: (i, 0)),
                  pl.BlockSpec((R, 128), lambda i: (i, 0)),
                  pl.BlockSpec((R, 256), lambda i: (i, 0)),
                  pl.BlockSpec((R, 16), lambda i: (i, 0)),
                  pl.BlockSpec((1, 256), lambda i: (0, 0))],
        out_specs=pl.BlockSpec((R, 256), lambda i: (i, 0)),
        out_shape=_SDS((N, 256), _f32),
    )(a0, a1, g, dinv16, b2r)


def kernel(x, edge_index, W1, b1, W2, b2):
    src = edge_index[0].astype(jnp.int32)
    dst = edge_index[1].astype(jnp.int32)
    src_rows = jnp.full((ECAP,), N, jnp.int32).at[:E].set(src).reshape(
        ROWS * 2, CHUNK // 2)
    dst_rows = jnp.full((ECAP,), N, jnp.int32).at[:E].set(dst).reshape(ROWS, CHUNK)
    x_pad = jnp.zeros((NP, 256), _f32).at[:N, :].set(x)
    ones_sm = jnp.ones((CHUNK, 128), _f32)
    zeros_big = jnp.zeros((NP, 128), _f32)
    b1r = b1.reshape(1, 512)
    b2r = b2.reshape(1, 256)

    cnt0, cnt1 = _deg_call(dst_rows, ones_sm, zeros_big)
    s0, s1, dinv16 = _k1(x_pad, cnt0, cnt1)
    a0, a1 = _agg_call(s0, s1, src_rows, dst_rows, zeros_big)
    g, t0, t1 = _k2(a0, a1, x_pad, dinv16, W1, b1r, W2)
    u0, u1 = _agg_call(t0, t1, src_rows, dst_rows, zeros_big)
    return _k3(u0, u1, g, dinv16, b2r)


# speedup vs baseline: 1.1629x; 1.1629x over previous
import functools

import jax
import jax.numpy as jnp
from jax import lax
from jax.experimental import pallas as pl
from jax.experimental.pallas import tpu as pltpu
from jax.experimental.pallas import tpu_sc as plsc

N = 10000
NP = 10240
E = 160000
CHUNK = 128
TILES = 16
AGG_CH = 80
DEG_CH = 40
ROWS = TILES * AGG_CH
ECAP = ROWS * CHUNK
NB = 2

_f32 = jnp.float32
_SDS = jax.ShapeDtypeStruct


def _mesh():
    return plsc.VectorSubcoreMesh(core_axis_name="c", subcore_axis_name="s")


def _deg_call(dst_rows, ones_sm, zeros_sm):
    @functools.partial(
        pl.kernel,
        out_type=(_SDS((NP, 128), _f32), _SDS((NP, 128), _f32)),
        scratch_types=[
            pltpu.VMEM((DEG_CH, CHUNK), jnp.int32),
            pltpu.VMEM((CHUNK, 128), _f32),
            pltpu.VMEM_SHARED((NP, 128), _f32),
        ],
        mesh=_mesh(),
    )
    def deg_k(dst_hbm, ones_hbm, z_hbm, cnt0, cnt1, idx_v, ones_v, acc):
        cid = lax.axis_index("c")
        sid = lax.axis_index("s")
        rows_per_tile = NP // TILES
        rbase = sid * rows_per_tile
        pltpu.sync_copy(z_hbm.at[pl.ds(rbase, rows_per_tile)],
                        acc.at[pl.ds(rbase, rows_per_tile)])
        pltpu.sync_copy(ones_hbm, ones_v)
        ebase = cid * (TILES * DEG_CH) + sid * DEG_CH
        pltpu.sync_copy(dst_hbm.at[pl.ds(ebase, DEG_CH)], idx_v)
        plsc.subcore_barrier()

        @pl.loop(0, DEG_CH)
        def _(j):
            pltpu.sync_copy(ones_v, acc.at[idx_v.at[j]], add=True)

        plsc.subcore_barrier()

        @pl.when(cid == 0)
        def _():
            pltpu.sync_copy(acc.at[pl.ds(rbase, rows_per_tile)],
                            cnt0.at[pl.ds(rbase, rows_per_tile)])

        @pl.when(cid == 1)
        def _():
            pltpu.sync_copy(acc.at[pl.ds(rbase, rows_per_tile)],
                            cnt1.at[pl.ds(rbase, rows_per_tile)])

    return deg_k(dst_rows, ones_sm, zeros_sm)


def _agg_call(s0, s1, src_rows, dst_rows, zeros_big):
    @functools.partial(
        pl.kernel,
        out_type=(_SDS((NP, 128), _f32), _SDS((NP, 128), _f32)),
        scratch_types=[
            pltpu.VMEM((AGG_CH // 2, CHUNK), jnp.int32),
            pltpu.VMEM((AGG_CH // 2, CHUNK), jnp.int32),
            pltpu.VMEM((NB, CHUNK, 128), _f32),
            pltpu.SemaphoreType.DMA((NB,)),
            pltpu.SemaphoreType.DMA((NB,)),
            pltpu.VMEM_SHARED((NP, 128), _f32),
        ],
        mesh=_mesh(),
    )
    def agg_k(s0_hbm, s1_hbm, sp_hbm, dp_hbm, z_hbm,
              out0, out1, sidx, didx, bufs, sem_g, sem_s, acc):
        cid = lax.axis_index("c")
        sid = lax.axis_index("s")
        rows_per_tile = NP // TILES
        rbase = sid * rows_per_tile
        ebase = sid * AGG_CH

        def pipeline(s_hbm, out_hbm):
            HC = AGG_CH // 2
            pltpu.sync_copy(sp_hbm.at[pl.ds(ebase, HC)], sidx)
            pltpu.sync_copy(dp_hbm.at[pl.ds(ebase, HC)], didx)
            for h in range(2):
                if h:
                    for b in range(NB):
                        pltpu.make_async_copy(
                            bufs.at[b], acc.at[didx.at[b]],
                            sem_s.at[b]).wait()
                    pltpu.sync_copy(sp_hbm.at[pl.ds(ebase + HC, HC)], sidx)
                    pltpu.sync_copy(dp_hbm.at[pl.ds(ebase + HC, HC)], didx)
                for b in range(NB):
                    pltpu.async_copy(s_hbm.at[sidx.at[b]], bufs.at[b],
                                     sem_g.at[b])
                if h == 0:
                    plsc.subcore_barrier()

                @pl.loop(0, HC // NB)
                def _(g):
                    for b in range(NB):
                        j = g * NB + b
                        pltpu.make_async_copy(
                            s_hbm.at[sidx.at[b]], bufs.at[b],
                            sem_g.at[b]).wait()
                        pltpu.async_copy(bufs.at[b], acc.at[didx.at[j]],
                                        sem_s.at[b], add=True)
                        nxt = j + NB

                        @pl.when(nxt < HC)
                        def _():
                            pltpu.make_async_copy(
                                bufs.at[b], acc.at[didx.at[j]],
                                sem_s.at[b]).wait()
                            pltpu.async_copy(s_hbm.at[sidx.at[nxt]],
                                             bufs.at[b], sem_g.at[b])

            for b in range(NB):
                pltpu.make_async_copy(
                    bufs.at[b], acc.at[didx.at[b]], sem_s.at[b]).wait()
            plsc.subcore_barrier()
            pltpu.sync_copy(acc.at[pl.ds(rbase, rows_per_tile)],
                            out_hbm.at[pl.ds(rbase, rows_per_tile)])

        pltpu.sync_copy(z_hbm.at[pl.ds(rbase, rows_per_tile)],
                        acc.at[pl.ds(rbase, rows_per_tile)])

        @pl.when(cid == 0)
        def _():
            pipeline(s0_hbm, out0)

        @pl.when(cid == 1)
        def _():
            pipeline(s1_hbm, out1)

    return agg_k(s0, s1, src_rows, dst_rows, zeros_big)


def _k1_body(x_ref, c0_ref, c1_ref, s0_ref, s1_ref, dinv_ref):
    cnt = c0_ref[...][:, :16] + c1_ref[...][:, :16]
    dinv16 = lax.rsqrt(cnt + 1.0)
    dinv = dinv16[:, :1]
    s = x_ref[...] * dinv
    s0_ref[...] = s[:, :128]
    s1_ref[...] = s[:, 128:]
    dinv_ref[...] = dinv16


def _k1(x_pad, cnt0, cnt1):
    R = 256
    return pl.pallas_call(
        _k1_body,
        grid=(NP // R,),
        in_specs=[pl.BlockSpec((R, 256), lambda i: (i, 0)),
                  pl.BlockSpec((R, 128), lambda i: (i, 0)),
                  pl.BlockSpec((R, 128), lambda i: (i, 0))],
        out_specs=[pl.BlockSpec((R, 128), lambda i: (i, 0)),
                   pl.BlockSpec((R, 128), lambda i: (i, 0)),
                   pl.BlockSpec((R, 16), lambda i: (i, 0))],
        out_shape=[_SDS((NP, 128), _f32), _SDS((NP, 128), _f32),
                   _SDS((NP, 16), _f32)],
    )(x_pad, cnt0, cnt1)


def _k2_body(a0_ref, a1_ref, x_ref, dinv_ref, w1_ref, b1_ref, w2_ref,
             g_ref, t0_ref, t1_ref):
    dinv = dinv_ref[...][:, :1]
    a = jnp.concatenate([a0_ref[...], a1_ref[...]], axis=1)
    t = dinv * a + (dinv * dinv) * x_ref[...]
    h1 = jnp.maximum(
        jnp.dot(t, w1_ref[...], preferred_element_type=_f32) + b1_ref[...],
        0.0)
    g = jnp.dot(h1, w2_ref[...], preferred_element_type=_f32)
    g_ref[...] = g
    s2 = g * dinv
    t0_ref[...] = s2[:, :128]
    t1_ref[...] = s2[:, 128:]


def _k2(a0, a1, x_pad, dinv16, W1, b1r, W2):
    R = 256
    return pl.pallas_call(
        _k2_body,
        grid=(NP // R,),
        in_specs=[pl.BlockSpec((R, 128), lambda i: (i, 0)),
                  pl.BlockSpec((R, 128), lambda i: (i, 0)),
                  pl.BlockSpec((R, 256), lambda i: (i, 0)),
                  pl.BlockSpec((R, 16), lambda i: (i, 0)),
                  pl.BlockSpec((256, 512), lambda i: (0, 0)),
                  pl.BlockSpec((1, 512), lambda i: (0, 0)),
                  pl.BlockSpec((512, 256), lambda i: (0, 0))],
        out_specs=[pl.BlockSpec((R, 256), lambda i: (i, 0)),
                   pl.BlockSpec((R, 128), lambda i: (i, 0)),
                   pl.BlockSpec((R, 128), lambda i: (i, 0))],
        out_shape=[_SDS((NP, 256), _f32), _SDS((NP, 128), _f32),
                   _SDS((NP, 128), _f32)],
    )(a0, a1, x_pad, dinv16, W1, b1r, W2)


def _k3_body(a0_ref, a1_ref, g_ref, dinv_ref, b2_ref, out_ref):
    dinv = dinv_ref[...][:, :1]
    a = jnp.concatenate([a0_ref[...], a1_ref[...]], axis=1)
    out_ref[...] = dinv * a + (dinv * dinv) * g_ref[...] + b2_ref[...]


def _k3(a0, a1, g, dinv16, b2r):
    R = 400
    return pl.pallas_call(
        _k3_body,
        grid=(N // R,),
        in_specs=[pl.BlockSpec((R, 128), lambda i: (i, 0)),
                  pl.BlockSpec((R, 128), lambda i: (i, 0)),
                  pl.BlockSpec((R, 256), lambda i: (i, 0)),
                  pl.BlockSpec((R, 16), lambda i: (i, 0)),
                  pl.BlockSpec((1, 256), lambda i: (0, 0))],
        out_specs=pl.BlockSpec((R, 256), lambda i: (i, 0)),
        out_shape=_SDS((N, 256), _f32),
    )(a0, a1, g, dinv16, b2r)


def kernel(x, edge_index, W1, b1, W2, b2):
    src = edge_index[0].astype(jnp.int32)
    dst = edge_index[1].astype(jnp.int32)
    src_rows = jnp.full((ECAP,), N, jnp.int32).at[:E].set(src).reshape(ROWS, CHUNK)
    dst_rows = jnp.full((ECAP,), N, jnp.int32).at[:E].set(dst).reshape(ROWS, CHUNK)
    x_pad = jnp.zeros((NP, 256), _f32).at[:N, :].set(x)
    ones_sm = jnp.ones((CHUNK, 128), _f32)
    zeros_big = jnp.zeros((NP, 128), _f32)
    b1r = b1.reshape(1, 512)
    b2r = b2.reshape(1, 256)

    cnt0, cnt1 = _deg_call(dst_rows, ones_sm, zeros_big)
    s0, s1, dinv16 = _k1(x_pad, cnt0, cnt1)
    a0, a1 = _agg_call(s0, s1, src_rows, dst_rows, zeros_big)
    g, t0, t1 = _k2(a0, a1, x_pad, dinv16, W1, b1r, W2)
    u0, u1 = _agg_call(t0, t1, src_rows, dst_rows, zeros_big)
    return _k3(u0, u1, g, dinv16, b2r)
